# Initial kernel scaffold; baseline (speedup 1.0000x reference)
import jax
import jax.numpy as jnp
from jax.experimental import pallas as pl


def kernel(x, edge_index, orders, W_mp, b_mp, W_out, b_out):
    raise NotImplementedError("write your pallas kernel here")


# speedup vs baseline: 11.3997x; 11.3997x over previous
import functools

import jax
import jax.numpy as jnp
from jax import lax
from jax.experimental import pallas as pl
from jax.experimental.pallas import tpu as pltpu
from jax.experimental.pallas import tpu_sc as plsc

N_NODES = 100000
K = 10
LANES = 16
NC = 2
NS = 16
NW = NC * NS
SW = 128
KSTREAMS = 8
JUNK = N_NODES
OUT_R = 100352
RPT = OUT_R // NS
ZCH = 128
NZ = RPT // ZCH
BR = 1000


def _make_step(ngroups: int):
    mesh = plsc.VectorSubcoreMesh(core_axis_name="c", subcore_axis_name="s")
    streams_per_tile = ngroups * KSTREAMS

    @functools.partial(
        pl.kernel,
        mesh=mesh,
        compiler_params=pltpu.CompilerParams(use_tc_tiling_on_sc=False),
        out_type=jax.ShapeDtypeStruct((NC, OUT_R, LANES), jnp.float32),
        scratch_types=[
            pltpu.VMEM((KSTREAMS, SW), jnp.int32),
            pltpu.VMEM((KSTREAMS, SW), jnp.int32),
            pltpu.VMEM((KSTREAMS, SW, LANES), jnp.float32),
            pltpu.VMEM((ZCH, LANES), jnp.float32),
            pltpu.VMEM_SHARED((OUT_R, LANES), jnp.float32),
            pltpu.SemaphoreType.DMA,
        ],
    )
    def step(h_hbm, src_hbm, dst_hbm, zrows_hbm, out_hbm,
             sidx, didx, rows, zbuf, acc, sem):
        cid = lax.axis_index("c")
        sid = lax.axis_index("s")
        pltpu.sync_copy(zrows_hbm, zbuf)
        zbase = sid * RPT
        for z in range(NZ):
            pltpu.sync_copy(zbuf, acc.at[pl.ds(zbase + z * ZCH, ZCH)])
        plsc.subcore_barrier()

        wid = cid * NS + sid
        row0 = wid * streams_per_tile

        def group(g, carry):
            base = row0 + g * KSTREAMS
            pltpu.sync_copy(src_hbm.at[pl.ds(base, KSTREAMS)], sidx)
            pltpu.sync_copy(dst_hbm.at[pl.ds(base, KSTREAMS)], didx)
            cps = [pltpu.async_copy(h_hbm.at[sidx.at[j]], rows.at[j], sem)
                   for j in range(KSTREAMS)]
            for cp in cps:
                cp.wait()
            for j in range(KSTREAMS):
                pltpu.sync_copy(rows.at[j], acc.at[didx.at[j]], add=True)
            return carry

        lax.fori_loop(0, ngroups, group, 0)
        plsc.subcore_barrier()
        pltpu.sync_copy(acc.at[pl.ds(zbase, RPT)],
                        out_hbm.at[cid, pl.ds(zbase, RPT)])

    return step


def _upd_body(h_ref, s_ref, a_ref, b2_ref, bias_ref, o_ref):
    h = h_ref[...]
    s = s_ref[0] + s_ref[1]
    deg = s[:, K:K + 1]
    ha = jnp.dot(h, a_ref[...], precision=lax.Precision.HIGHEST,
                 preferred_element_type=jnp.float32)
    sb = jnp.dot(s, b2_ref[...], precision=lax.Precision.HIGHEST,
                 preferred_element_type=jnp.float32)
    o_ref[...] = h + deg * (ha + bias_ref[...]) + sb


def _update(h, s2, amat, b2mat, bias):
    return pl.pallas_call(
        _upd_body,
        grid=(N_NODES // BR,),
        in_specs=[
            pl.BlockSpec((BR, LANES), lambda i: (i, 0)),
            pl.BlockSpec((NC, BR, LANES), lambda i: (0, i, 0)),
            pl.BlockSpec((LANES, LANES), lambda i: (0, 0)),
            pl.BlockSpec((LANES, LANES), lambda i: (0, 0)),
            pl.BlockSpec((1, LANES), lambda i: (0, 0)),
        ],
        out_specs=pl.BlockSpec((BR, LANES), lambda i: (i, 0)),
        out_shape=jax.ShapeDtypeStruct((N_NODES, LANES), jnp.float32),
    )(h, s2, amat, b2mat, bias)


def _final_body(h_ref, w_ref, b_ref, o_ref):
    o_ref[...] = (jnp.dot(h_ref[...], w_ref[...],
                          precision=lax.Precision.HIGHEST,
                          preferred_element_type=jnp.float32) + b_ref[...])


def _final(h, wo, bo):
    return pl.pallas_call(
        _final_body,
        grid=(N_NODES // BR,),
        in_specs=[
            pl.BlockSpec((BR, LANES), lambda i: (i, 0)),
            pl.BlockSpec((LANES, LANES), lambda i: (0, 0)),
            pl.BlockSpec((1, LANES), lambda i: (0, 0)),
        ],
        out_specs=pl.BlockSpec((BR, LANES), lambda i: (i, 0)),
        out_shape=jax.ShapeDtypeStruct((N_NODES, LANES), jnp.float32),
    )(h, wo, bo)


def kernel(x, edge_index, orders, W_mp, b_mp, W_out, b_out):
    n_orders, n_edges = orders.shape
    ei = edge_index.astype(jnp.int32)
    ords = orders.astype(jnp.int32)
    src = ei[1][ords]
    dst = ei[0][ords]

    nstreams = pl.cdiv(n_edges, SW)
    per_tile = KSTREAMS * NW
    nstreams_pad = pl.cdiv(nstreams, per_tile) * per_tile
    ngroups = nstreams_pad // per_tile
    e_pad = nstreams_pad * SW - n_edges
    src_p = jnp.pad(src, ((0, 0), (0, e_pad))).reshape(n_orders, -1, SW)
    dst_p = jnp.pad(dst, ((0, 0), (0, e_pad)),
                    constant_values=JUNK).reshape(n_orders, -1, SW)

    w1 = W_mp[:, :K]
    w2 = W_mp[:, K:]
    amat = jnp.zeros((LANES, LANES), jnp.float32).at[:K, :K].set((w1 - w2).T)
    b2mat = jnp.zeros((LANES, LANES), jnp.float32).at[:K, :K].set(w2.T)
    bias = jnp.zeros((1, LANES), jnp.float32).at[0, :K].set(b_mp)
    wo = jnp.zeros((LANES, LANES), jnp.float32).at[:K, :W_out.shape[0]].set(W_out.T)
    bo = jnp.zeros((1, LANES), jnp.float32).at[0, :W_out.shape[0]].set(b_out)
    zrows = jnp.zeros((ZCH, LANES), jnp.float32)

    h = jnp.concatenate(
        [x, jnp.ones((N_NODES, 1), jnp.float32),
         jnp.zeros((N_NODES, LANES - K - 1), jnp.float32)], axis=1)

    step = _make_step(ngroups)
    for i in range(n_orders):
        s2 = step(h, src_p[i], dst_p[i], zrows)
        h = _update(h, s2, amat, b2mat, bias)
    out = _final(h, wo, bo)
    return out[:, :W_out.shape[0]]


# speedup vs baseline: 14.7336x; 1.2925x over previous
import functools

import jax
import jax.numpy as jnp
from jax import lax
from jax.experimental import pallas as pl
from jax.experimental.pallas import tpu as pltpu
from jax.experimental.pallas import tpu_sc as plsc

N_NODES = 100000
K = 10
LANES = 16
NC = 2
NS = 16
NW = NC * NS
SW = 128
KSTREAMS = 8
JUNK = N_NODES
OUT_R = 100352
RPT = OUT_R // NS
ZCH = 128
NZ = RPT // ZCH
BR = 1000


def _make_step(ngroups: int):
    mesh = plsc.VectorSubcoreMesh(core_axis_name="c", subcore_axis_name="s")
    streams_per_tile = ngroups * KSTREAMS

    @functools.partial(
        pl.kernel,
        mesh=mesh,
        compiler_params=pltpu.CompilerParams(use_tc_tiling_on_sc=False),
        out_type=jax.ShapeDtypeStruct((NC, OUT_R, LANES), jnp.float32),
        scratch_types=[
            pltpu.VMEM((KSTREAMS, SW), jnp.int32),
            pltpu.VMEM((KSTREAMS, SW), jnp.int32),
            pltpu.VMEM((KSTREAMS, SW), jnp.int32),
            pltpu.VMEM((KSTREAMS, SW, LANES), jnp.float32),
            pltpu.VMEM((ZCH, LANES), jnp.float32),
            pltpu.VMEM_SHARED((OUT_R, LANES), jnp.float32),
            pltpu.SemaphoreType.DMA,
        ],
    )
    def step(h_hbm, ord_hbm, srcall_hbm, dstall_hbm, zrows_hbm, out_hbm,
             oidx, sidx, didx, rows, zbuf, acc, sem):
        cid = lax.axis_index("c")
        sid = lax.axis_index("s")
        pltpu.sync_copy(zrows_hbm, zbuf)
        zbase = sid * RPT
        for z in range(NZ):
            pltpu.sync_copy(zbuf, acc.at[pl.ds(zbase + z * ZCH, ZCH)])
        plsc.subcore_barrier()

        wid = cid * NS + sid
        row0 = wid * streams_per_tile

        def group(g, carry):
            base = row0 + g * KSTREAMS
            pltpu.sync_copy(ord_hbm.at[pl.ds(base, KSTREAMS)], oidx)
            cps = [pltpu.async_copy(srcall_hbm.at[oidx.at[j]], sidx.at[j], sem)
                   for j in range(KSTREAMS)]
            cps += [pltpu.async_copy(dstall_hbm.at[oidx.at[j]], didx.at[j], sem)
                    for j in range(KSTREAMS)]
            for cp in cps:
                cp.wait()
            cps = [pltpu.async_copy(h_hbm.at[sidx.at[j]], rows.at[j], sem)
                   for j in range(KSTREAMS)]
            for cp in cps:
                cp.wait()
            for j in range(KSTREAMS):
                pltpu.sync_copy(rows.at[j], acc.at[didx.at[j]], add=True)
            return carry

        lax.fori_loop(0, ngroups, group, 0)
        plsc.subcore_barrier()
        pltpu.sync_copy(acc.at[pl.ds(zbase, RPT)],
                        out_hbm.at[cid, pl.ds(zbase, RPT)])

    return step


def _upd_body(h_ref, s_ref, a_ref, b2_ref, bias_ref, o_ref):
    h = h_ref[...]
    s = s_ref[0] + s_ref[1]
    deg = s[:, K:K + 1]
    ha = jnp.dot(h, a_ref[...], precision=lax.Precision.HIGHEST,
                 preferred_element_type=jnp.float32)
    sb = jnp.dot(s, b2_ref[...], precision=lax.Precision.HIGHEST,
                 preferred_element_type=jnp.float32)
    o_ref[...] = h + deg * (ha + bias_ref[...]) + sb


def _update(h, s2, amat, b2mat, bias):
    return pl.pallas_call(
        _upd_body,
        grid=(N_NODES // BR,),
        in_specs=[
            pl.BlockSpec((BR, LANES), lambda i: (i, 0)),
            pl.BlockSpec((NC, BR, LANES), lambda i: (0, i, 0)),
            pl.BlockSpec((LANES, LANES), lambda i: (0, 0)),
            pl.BlockSpec((LANES, LANES), lambda i: (0, 0)),
            pl.BlockSpec((1, LANES), lambda i: (0, 0)),
        ],
        out_specs=pl.BlockSpec((BR, LANES), lambda i: (i, 0)),
        out_shape=jax.ShapeDtypeStruct((N_NODES, LANES), jnp.float32),
    )(h, s2, amat, b2mat, bias)


def _final_body(h_ref, w_ref, b_ref, o_ref):
    o_ref[...] = (jnp.dot(h_ref[...], w_ref[...],
                          precision=lax.Precision.HIGHEST,
                          preferred_element_type=jnp.float32) + b_ref[...])


def _final(h, wo, bo):
    return pl.pallas_call(
        _final_body,
        grid=(N_NODES // BR,),
        in_specs=[
            pl.BlockSpec((BR, LANES), lambda i: (i, 0)),
            pl.BlockSpec((LANES, LANES), lambda i: (0, 0)),
            pl.BlockSpec((1, LANES), lambda i: (0, 0)),
        ],
        out_specs=pl.BlockSpec((BR, LANES), lambda i: (i, 0)),
        out_shape=jax.ShapeDtypeStruct((N_NODES, LANES), jnp.float32),
    )(h, wo, bo)


def kernel(x, edge_index, orders, W_mp, b_mp, W_out, b_out):
    n_orders, n_edges = orders.shape
    n_all = edge_index.shape[1]
    ei = edge_index.astype(jnp.int32)
    ords = orders.astype(jnp.int32)
    srcall = jnp.pad(ei[1], (0, 8))
    dstall = jnp.pad(ei[0], (0, 8), constant_values=JUNK)

    nstreams = pl.cdiv(n_edges, SW)
    per_tile = KSTREAMS * NW
    nstreams_pad = pl.cdiv(nstreams, per_tile) * per_tile
    ngroups = nstreams_pad // per_tile
    e_pad = nstreams_pad * SW - n_edges
    ords_p = jnp.pad(ords, ((0, 0), (0, e_pad)),
                     constant_values=n_all).reshape(n_orders, -1, SW)

    w1 = W_mp[:, :K]
    w2 = W_mp[:, K:]
    amat = jnp.zeros((LANES, LANES), jnp.float32).at[:K, :K].set((w1 - w2).T)
    b2mat = jnp.zeros((LANES, LANES), jnp.float32).at[:K, :K].set(w2.T)
    bias = jnp.zeros((1, LANES), jnp.float32).at[0, :K].set(b_mp)
    wo = jnp.zeros((LANES, LANES), jnp.float32).at[:K, :W_out.shape[0]].set(W_out.T)
    bo = jnp.zeros((1, LANES), jnp.float32).at[0, :W_out.shape[0]].set(b_out)
    zrows = jnp.zeros((ZCH, LANES), jnp.float32)

    h = jnp.concatenate(
        [x, jnp.ones((N_NODES, 1), jnp.float32),
         jnp.zeros((N_NODES, LANES - K - 1), jnp.float32)], axis=1)

    step = _make_step(ngroups)
    for i in range(n_orders):
        s2 = step(h, ords_p[i], srcall, dstall, zrows)
        h = _update(h, s2, amat, b2mat, bias)
    out = _final(h, wo, bo)
    return out[:, :W_out.shape[0]]
